# baseline (speedup 1.0000x reference)
import functools

import jax
import jax.numpy as jnp
from jax import lax
from jax.experimental import pallas as pl
from jax.experimental.pallas import tpu as pltpu
from jax.experimental.pallas import tpu_sc as plsc

EMB = 768
E = 8
K = 2
HID = 4 * EMB
S = 2048

T = 256
NT = (K * S) // T + E
NSLOT = NT * T
NH = 2
H_BLK = HID // NH

SW_W = 128
NW = 32
TOK_W = S // NW


def _router_body(x_ref, wr_ref, br_ref,
                 p0_ref, p1_ref, w0_ref, w1_ref, te_ref, tv_ref,
                 sw0_ref, sw1_ref):
    xb = x_ref[...]
    logits = jnp.dot(xb, wr_ref[...], preferred_element_type=jnp.float32)
    logits = logits + br_ref[...]
    eidx = lax.broadcasted_iota(jnp.int32, (S, E), 1)
    m1 = jnp.max(logits, axis=-1, keepdims=True)
    i1 = jnp.min(jnp.where(logits == m1, eidx, E), axis=-1, keepdims=True)
    l2 = jnp.where(eidx == i1, -jnp.inf, logits)
    m2 = jnp.max(l2, axis=-1, keepdims=True)
    i2 = jnp.min(jnp.where(l2 == m2, eidx, E), axis=-1, keepdims=True)
    t = jnp.exp(m2 - m1)
    wa = 1.0 / (1.0 + t)
    wb = t / (1.0 + t)
    w0_ref[...] = wa
    w1_ref[...] = wb
    lane0 = lax.broadcasted_iota(jnp.int32, (S, SW_W), 1) == 0
    sw0_ref[...] = jnp.where(lane0, wa, 0.0)
    sw1_ref[...] = jnp.where(lane0, wb, 0.0)

    h1 = (eidx == i1).astype(jnp.float32)
    h2 = (eidx == i2).astype(jnp.float32)
    cnt = h1 + h2
    inc = cnt
    d = 1
    while d < S:
        shifted = jnp.concatenate(
            [jnp.zeros((d, E), jnp.float32), inc[:-d, :]], axis=0)
        inc = inc + shifted
        d *= 2
    excl = inc - cnt
    counts = jnp.max(inc, axis=0, keepdims=True)
    padded = jnp.ceil(counts / T) * T
    ri = lax.broadcasted_iota(jnp.int32, (E, E), 0)
    ci = lax.broadcasted_iota(jnp.int32, (E, E), 1)
    m_lt = (ri < ci).astype(jnp.float32)
    base = jnp.dot(padded, m_lt, preferred_element_type=jnp.float32)
    cumpad = base + padded

    r1 = jnp.sum(excl * h1, axis=1, keepdims=True)
    b1 = jnp.sum(base * h1, axis=1, keepdims=True)
    p0_ref[...] = (b1 + r1).astype(jnp.int32)
    r2 = jnp.sum(excl * h2, axis=1, keepdims=True)
    b2 = jnp.sum(base * h2, axis=1, keepdims=True)
    p1_ref[...] = (b2 + r2).astype(jnp.int32)

    tstart = (lax.broadcasted_iota(jnp.int32, (NT, E), 0) * T).astype(
        jnp.float32)
    cmp = (tstart >= cumpad).astype(jnp.int32)
    te = jnp.sum(cmp, axis=1, keepdims=True)
    te_ref[...] = jnp.minimum(te, E - 1)
    total = jnp.sum(padded)
    tv_ref[...] = (tstart[:, 0:1] < total).astype(jnp.int32)


def _router_call(x2d, Wr, br2d, interpret=False):
    return pl.pallas_call(
        _router_body,
        in_specs=[
            pl.BlockSpec((S, EMB), lambda: (0, 0)),
            pl.BlockSpec((EMB, E), lambda: (0, 0)),
            pl.BlockSpec((1, E), lambda: (0, 0)),
        ],
        out_specs=[
            pl.BlockSpec((S, 1), lambda: (0, 0)),
            pl.BlockSpec((S, 1), lambda: (0, 0)),
            pl.BlockSpec((S, 1), lambda: (0, 0)),
            pl.BlockSpec((S, 1), lambda: (0, 0)),
            pl.BlockSpec((NT, 1), lambda: (0, 0)),
            pl.BlockSpec((NT, 1), lambda: (0, 0)),
            pl.BlockSpec((S, SW_W), lambda: (0, 0)),
            pl.BlockSpec((S, SW_W), lambda: (0, 0)),
        ],
        out_shape=[
            jax.ShapeDtypeStruct((S, 1), jnp.int32),
            jax.ShapeDtypeStruct((S, 1), jnp.int32),
            jax.ShapeDtypeStruct((S, 1), jnp.float32),
            jax.ShapeDtypeStruct((S, 1), jnp.float32),
            jax.ShapeDtypeStruct((NT, 1), jnp.int32),
            jax.ShapeDtypeStruct((NT, 1), jnp.int32),
            jax.ShapeDtypeStruct((S, SW_W), jnp.float32),
            jax.ShapeDtypeStruct((S, SW_W), jnp.float32),
        ],
        interpret=interpret,
    )(x2d, Wr, br2d)


def _dispatch_body(x_hbm, p0_hbm, p1_hbm, sw0_hbm, sw1_hbm,
                   xg_hbm, sw_hbm, rows_v, dest_v, swv, sem):
    wid = lax.axis_index("s") * 2 + lax.axis_index("c")
    base = wid * TOK_W
    pltpu.sync_copy(x_hbm.at[pl.ds(base, TOK_W)], rows_v)
    for k in range(K):
        p_hbm = p0_hbm if k == 0 else p1_hbm
        w_hbm = sw0_hbm if k == 0 else sw1_hbm
        pltpu.sync_copy(p_hbm.at[pl.ds(base, TOK_W)], dest_v)
        pltpu.sync_copy(w_hbm.at[pl.ds(base, TOK_W)], swv)
        pltpu.async_copy(rows_v, xg_hbm.at[dest_v], sem).wait()
        pltpu.async_copy(swv, sw_hbm.at[dest_v], sem).wait()


@functools.cache
def _dispatch_call():
    return pl.kernel(
        _dispatch_body,
        mesh=plsc.VectorSubcoreMesh(core_axis_name="c", subcore_axis_name="s"),
        out_type=[
            jax.ShapeDtypeStruct((NSLOT, EMB), jnp.float32),
            jax.ShapeDtypeStruct((NSLOT, SW_W), jnp.float32),
        ],
        scratch_types=[
            pltpu.VMEM((TOK_W, EMB), jnp.float32),
            pltpu.VMEM((TOK_W,), jnp.int32),
            pltpu.VMEM((TOK_W, SW_W), jnp.float32),
            pltpu.SemaphoreType.DMA,
        ],
    )


def _ffn_body(te_ref, tv_ref, xg_ref, sw_ref, w1_ref, w3_ref, w2_ref,
              out_ref):
    t = pl.program_id(0)
    h = pl.program_id(1)

    @pl.when(tv_ref[t] == 1)
    def _():
        xb = xg_ref[...].astype(jnp.bfloat16)
        w1b = w1_ref[0].astype(jnp.bfloat16)
        w3b = w3_ref[0].astype(jnp.bfloat16)
        w2b = w2_ref[0].astype(jnp.bfloat16)
        a = jnp.dot(xb, w1b, preferred_element_type=jnp.float32)
        g = jnp.dot(xb, w3b, preferred_element_type=jnp.float32)
        hh = (a * jax.nn.sigmoid(a)) * g
        part = jnp.dot(hh.astype(jnp.bfloat16), w2b,
                       preferred_element_type=jnp.float32)
        contrib = part * sw_ref[:, 0:1]

        @pl.when(h == 0)
        def _():
            out_ref[...] = contrib

        @pl.when(h != 0)
        def _():
            out_ref[...] += contrib


def _zig(t, h):
    return jnp.where(t % 2 == 0, h, NH - 1 - h)


def _ffn_call(te, tv, xg, sw, W1, W3, W2, interpret=False):
    grid_spec = pltpu.PrefetchScalarGridSpec(
        num_scalar_prefetch=2,
        grid=(NT, NH),
        in_specs=[
            pl.BlockSpec((T, EMB), lambda t, h, te, tv: (t, 0)),
            pl.BlockSpec((T, SW_W), lambda t, h, te, tv: (t, 0)),
            pl.BlockSpec((1, EMB, H_BLK),
                         lambda t, h, te, tv: (te[t], 0, _zig(t, h))),
            pl.BlockSpec((1, EMB, H_BLK),
                         lambda t, h, te, tv: (te[t], 0, _zig(t, h))),
            pl.BlockSpec((1, H_BLK, EMB),
                         lambda t, h, te, tv: (te[t], _zig(t, h), 0)),
        ],
        out_specs=pl.BlockSpec((T, EMB), lambda t, h, te, tv: (t, 0)),
    )
    return pl.pallas_call(
        _ffn_body,
        grid_spec=grid_spec,
        out_shape=jax.ShapeDtypeStruct((NSLOT, EMB), jnp.float32),
        interpret=interpret,
    )(te, tv, xg, sw, W1, W3, W2)


def _combine_body(yg_hbm, p0_hbm, p1_hbm, out_hbm, r0, r1, pv, sem):
    wid = lax.axis_index("s") * 2 + lax.axis_index("c")
    base = wid * TOK_W
    pltpu.sync_copy(p0_hbm.at[pl.ds(base, TOK_W)], pv)
    pltpu.async_copy(yg_hbm.at[pv], r0, sem).wait()
    pltpu.sync_copy(p1_hbm.at[pl.ds(base, TOK_W)], pv)
    pltpu.async_copy(yg_hbm.at[pv], r1, sem).wait()

    def row_body(r, carry):
        for c in range(EMB // 16):
            sl = pl.ds(c * 16, 16)
            r0[r, sl] += r1[r, sl]
        return carry

    lax.fori_loop(0, TOK_W, row_body, 0)
    pltpu.sync_copy(r0, out_hbm.at[pl.ds(base, TOK_W)])


@functools.cache
def _combine_call():
    return pl.kernel(
        _combine_body,
        mesh=plsc.VectorSubcoreMesh(core_axis_name="c", subcore_axis_name="s"),
        out_type=jax.ShapeDtypeStruct((S, EMB), jnp.float32),
        scratch_types=[
            pltpu.VMEM((TOK_W, EMB), jnp.float32),
            pltpu.VMEM((TOK_W, EMB), jnp.float32),
            pltpu.VMEM((TOK_W,), jnp.int32),
            pltpu.SemaphoreType.DMA,
        ],
    )


@jax.jit
def kernel(x, Wr, br, W1, W3, W2):
    x2d = x.reshape(S, EMB)
    p0, p1, w0, w1, te, tv, sw0, sw1 = _router_call(x2d, Wr, br.reshape(1, E))
    p0f = p0.reshape(S)
    p1f = p1.reshape(S)
    xg, sw = _dispatch_call()(x2d, p0f, p1f, sw0, sw1)
    yg = _ffn_call(te.reshape(NT), tv.reshape(NT), xg, sw, W1, W3, W2)
    out = _combine_call()(yg, p0f, p1f)
    return out.reshape(1, S, EMB)


# speedup vs baseline: 1.1724x; 1.1724x over previous
import functools

import jax
import jax.numpy as jnp
from jax import lax
from jax.experimental import pallas as pl
from jax.experimental.pallas import tpu as pltpu
from jax.experimental.pallas import tpu_sc as plsc

EMB = 768
E = 8
K = 2
HID = 4 * EMB
S = 2048

T = 1024
NT = (K * S) // T + E
NSLOT = NT * T
NH = 3
H_BLK = HID // NH

SW_W = 128
NW = 32
TOK_W = S // NW


def _router_body(x_ref, wr_ref, br_ref,
                 p0_ref, p1_ref, w0_ref, w1_ref, te_ref, tv_ref,
                 sw0_ref, sw1_ref):
    xb = x_ref[...]
    logits = jnp.dot(xb, wr_ref[...], preferred_element_type=jnp.float32)
    logits = logits + br_ref[...]
    eidx = lax.broadcasted_iota(jnp.int32, (S, E), 1)
    m1 = jnp.max(logits, axis=-1, keepdims=True)
    i1 = jnp.min(jnp.where(logits == m1, eidx, E), axis=-1, keepdims=True)
    l2 = jnp.where(eidx == i1, -jnp.inf, logits)
    m2 = jnp.max(l2, axis=-1, keepdims=True)
    i2 = jnp.min(jnp.where(l2 == m2, eidx, E), axis=-1, keepdims=True)
    t = jnp.exp(m2 - m1)
    wa = 1.0 / (1.0 + t)
    wb = t / (1.0 + t)
    w0_ref[...] = wa
    w1_ref[...] = wb
    lane0 = lax.broadcasted_iota(jnp.int32, (S, SW_W), 1) == 0
    sw0_ref[...] = jnp.where(lane0, wa, 0.0)
    sw1_ref[...] = jnp.where(lane0, wb, 0.0)

    h1 = (eidx == i1).astype(jnp.float32)
    h2 = (eidx == i2).astype(jnp.float32)
    cnt = h1 + h2
    inc = cnt
    d = 1
    while d < S:
        shifted = jnp.concatenate(
            [jnp.zeros((d, E), jnp.float32), inc[:-d, :]], axis=0)
        inc = inc + shifted
        d *= 2
    excl = inc - cnt
    counts = jnp.max(inc, axis=0, keepdims=True)
    padded = jnp.ceil(counts / T) * T
    ri = lax.broadcasted_iota(jnp.int32, (E, E), 0)
    ci = lax.broadcasted_iota(jnp.int32, (E, E), 1)
    m_lt = (ri < ci).astype(jnp.float32)
    base = jnp.dot(padded, m_lt, preferred_element_type=jnp.float32)
    cumpad = base + padded

    r1 = jnp.sum(excl * h1, axis=1, keepdims=True)
    b1 = jnp.sum(base * h1, axis=1, keepdims=True)
    p0_ref[...] = (b1 + r1).astype(jnp.int32)
    r2 = jnp.sum(excl * h2, axis=1, keepdims=True)
    b2 = jnp.sum(base * h2, axis=1, keepdims=True)
    p1_ref[...] = (b2 + r2).astype(jnp.int32)

    tstart = (lax.broadcasted_iota(jnp.int32, (NT, E), 0) * T).astype(
        jnp.float32)
    cmp = (tstart >= cumpad).astype(jnp.int32)
    te = jnp.sum(cmp, axis=1, keepdims=True)
    total = jnp.sum(padded)
    e_last = jnp.sum((cumpad < total).astype(jnp.int32), axis=1, keepdims=True)
    valid = tstart[:, 0:1] < total
    te_ref[...] = jnp.where(valid, jnp.minimum(te, E - 1), e_last)
    tv_ref[...] = valid.astype(jnp.int32)


def _router_call(x2d, Wr, br2d, interpret=False):
    return pl.pallas_call(
        _router_body,
        in_specs=[
            pl.BlockSpec((S, EMB), lambda: (0, 0)),
            pl.BlockSpec((EMB, E), lambda: (0, 0)),
            pl.BlockSpec((1, E), lambda: (0, 0)),
        ],
        out_specs=[
            pl.BlockSpec((S, 1), lambda: (0, 0)),
            pl.BlockSpec((S, 1), lambda: (0, 0)),
            pl.BlockSpec((S, 1), lambda: (0, 0)),
            pl.BlockSpec((S, 1), lambda: (0, 0)),
            pl.BlockSpec((NT, 1), lambda: (0, 0)),
            pl.BlockSpec((NT, 1), lambda: (0, 0)),
            pl.BlockSpec((S, SW_W), lambda: (0, 0)),
            pl.BlockSpec((S, SW_W), lambda: (0, 0)),
        ],
        out_shape=[
            jax.ShapeDtypeStruct((S, 1), jnp.int32),
            jax.ShapeDtypeStruct((S, 1), jnp.int32),
            jax.ShapeDtypeStruct((S, 1), jnp.float32),
            jax.ShapeDtypeStruct((S, 1), jnp.float32),
            jax.ShapeDtypeStruct((NT, 1), jnp.int32),
            jax.ShapeDtypeStruct((NT, 1), jnp.int32),
            jax.ShapeDtypeStruct((S, SW_W), jnp.float32),
            jax.ShapeDtypeStruct((S, SW_W), jnp.float32),
        ],
        interpret=interpret,
    )(x2d, Wr, br2d)


def _dispatch_body(x_hbm, p0_hbm, p1_hbm, sw0_hbm, sw1_hbm,
                   xg_hbm, sw_hbm, rows_v, dest_v, swv, sem):
    wid = lax.axis_index("s") * 2 + lax.axis_index("c")
    base = wid * TOK_W
    pltpu.sync_copy(x_hbm.at[pl.ds(base, TOK_W)], rows_v)
    for k in range(K):
        p_hbm = p0_hbm if k == 0 else p1_hbm
        w_hbm = sw0_hbm if k == 0 else sw1_hbm
        pltpu.sync_copy(p_hbm.at[pl.ds(base, TOK_W)], dest_v)
        pltpu.sync_copy(w_hbm.at[pl.ds(base, TOK_W)], swv)
        pltpu.async_copy(rows_v, xg_hbm.at[dest_v], sem).wait()
        pltpu.async_copy(swv, sw_hbm.at[dest_v], sem).wait()


@functools.cache
def _dispatch_call():
    return pl.kernel(
        _dispatch_body,
        mesh=plsc.VectorSubcoreMesh(core_axis_name="c", subcore_axis_name="s"),
        out_type=[
            jax.ShapeDtypeStruct((NSLOT, EMB), jnp.float32),
            jax.ShapeDtypeStruct((NSLOT, SW_W), jnp.float32),
        ],
        scratch_types=[
            pltpu.VMEM((TOK_W, EMB), jnp.float32),
            pltpu.VMEM((TOK_W,), jnp.int32),
            pltpu.VMEM((TOK_W, SW_W), jnp.float32),
            pltpu.SemaphoreType.DMA,
        ],
    )


def _ffn_body(te_ref, tv_ref, xg_ref, sw_ref, w1_ref, w3_ref, w2_ref,
              out_ref):
    t = pl.program_id(0)
    h = pl.program_id(1)

    @pl.when(tv_ref[t] == 1)
    def _():
        xb = xg_ref[...].astype(jnp.bfloat16)
        w1b = w1_ref[0].astype(jnp.bfloat16)
        w3b = w3_ref[0].astype(jnp.bfloat16)
        w2b = w2_ref[0].astype(jnp.bfloat16)
        a = jnp.dot(xb, w1b, preferred_element_type=jnp.float32)
        g = jnp.dot(xb, w3b, preferred_element_type=jnp.float32)
        hh = (a * jax.nn.sigmoid(a)) * g
        part = jnp.dot(hh.astype(jnp.bfloat16), w2b,
                       preferred_element_type=jnp.float32)
        contrib = part * sw_ref[:, 0:1]

        @pl.when(h == 0)
        def _():
            out_ref[...] = contrib

        @pl.when(h != 0)
        def _():
            out_ref[...] += contrib


def _zig(t, h, tv):
    return jnp.where(tv[t] == 1, jnp.where(t % 2 == 0, h, NH - 1 - h), 0)


def _ffn_call(te, tv, xg, sw, W1, W3, W2, interpret=False):
    grid_spec = pltpu.PrefetchScalarGridSpec(
        num_scalar_prefetch=2,
        grid=(NT, NH),
        in_specs=[
            pl.BlockSpec((T, EMB), lambda t, h, te, tv: (t, 0)),
            pl.BlockSpec((T, SW_W), lambda t, h, te, tv: (t, 0)),
            pl.BlockSpec((1, EMB, H_BLK),
                         lambda t, h, te, tv: (te[t], 0, _zig(t, h, tv))),
            pl.BlockSpec((1, EMB, H_BLK),
                         lambda t, h, te, tv: (te[t], 0, _zig(t, h, tv))),
            pl.BlockSpec((1, H_BLK, EMB),
                         lambda t, h, te, tv: (te[t], _zig(t, h, tv), 0)),
        ],
        out_specs=pl.BlockSpec((T, EMB), lambda t, h, te, tv: (t, 0)),
    )
    return pl.pallas_call(
        _ffn_body,
        grid_spec=grid_spec,
        out_shape=jax.ShapeDtypeStruct((NSLOT, EMB), jnp.float32),
        interpret=interpret,
    )(te, tv, xg, sw, W1, W3, W2)


def _combine_body(yg_hbm, p0_hbm, p1_hbm, out_hbm, r0, r1, pv, sem):
    wid = lax.axis_index("s") * 2 + lax.axis_index("c")
    base = wid * TOK_W
    pltpu.sync_copy(p0_hbm.at[pl.ds(base, TOK_W)], pv)
    pltpu.async_copy(yg_hbm.at[pv], r0, sem).wait()
    pltpu.sync_copy(p1_hbm.at[pl.ds(base, TOK_W)], pv)
    pltpu.async_copy(yg_hbm.at[pv], r1, sem).wait()

    def row_body(r, carry):
        for c in range(EMB // 16):
            sl = pl.ds(c * 16, 16)
            r0[r, sl] += r1[r, sl]
        return carry

    lax.fori_loop(0, TOK_W, row_body, 0)
    pltpu.sync_copy(r0, out_hbm.at[pl.ds(base, TOK_W)])


@functools.cache
def _combine_call():
    return pl.kernel(
        _combine_body,
        mesh=plsc.VectorSubcoreMesh(core_axis_name="c", subcore_axis_name="s"),
        out_type=jax.ShapeDtypeStruct((S, EMB), jnp.float32),
        scratch_types=[
            pltpu.VMEM((TOK_W, EMB), jnp.float32),
            pltpu.VMEM((TOK_W, EMB), jnp.float32),
            pltpu.VMEM((TOK_W,), jnp.int32),
            pltpu.SemaphoreType.DMA,
        ],
    )


@jax.jit
def kernel(x, Wr, br, W1, W3, W2):
    x2d = x.reshape(S, EMB)
    p0, p1, w0, w1, te, tv, sw0, sw1 = _router_call(x2d, Wr, br.reshape(1, E))
    p0f = p0.reshape(S)
    p1f = p1.reshape(S)
    xg, sw = _dispatch_call()(x2d, p0f, p1f, sw0, sw1)
    yg = _ffn_call(te.reshape(NT), tv.reshape(NT), xg, sw, W1, W3, W2)
    out = _combine_call()(yg, p0f, p1f)
    return out.reshape(1, S, EMB)
